# baseline (speedup 1.0000x reference)
import functools

import jax
import jax.numpy as jnp
from jax import lax
from jax.experimental import pallas as pl
from jax.experimental.pallas import tpu as pltpu
from jax.experimental.pallas import tpu_sc as plsc

EMB = 64
BATCH = 16384

NC = 2
NS = 16
NW = NC * NS
BPW = BATCH // NW
ICHUNK = 128
NCHUNK = BPW // ICHUNK


def _mf_body(u_id_hbm, i_id_hbm, ut_hbm, it_hbm, out_hbm,
             uorig_v, iorig_v, uphys_v, iphys_v, urows_v, irows_v, outb_v, sem):
    wid = lax.axis_index("s") * NC + lax.axis_index("c")

    pltpu.sync_copy(u_id_hbm.at[wid], uorig_v)
    pltpu.sync_copy(i_id_hbm.at[wid], iorig_v)
    for c in range(NCHUNK):
        for k in range(ICHUNK // 16):
            s = pl.ds(k * 16, 16)
            uphys_v[c, s] = jnp.right_shift(uorig_v[c, s], 1)
            iphys_v[c, s] = jnp.right_shift(iorig_v[c, s], 1)

    lane = lax.iota(jnp.int32, 16)

    for h in range(2):
        copies = []
        for cc in range(NCHUNK // 2):
            c = h * (NCHUNK // 2) + cc
            copies.append(pltpu.async_copy(
                ut_hbm.at[uphys_v.at[c]],
                urows_v.at[pl.ds(cc * ICHUNK, ICHUNK)], sem))
            copies.append(pltpu.async_copy(
                it_hbm.at[iphys_v.at[c]],
                irows_v.at[pl.ds(cc * ICHUNK, ICHUNK)], sem))
        for cp in copies:
            cp.wait()

        def body(ch, _):
            a00 = jnp.zeros((16,), jnp.float32)
            a01 = jnp.zeros((16,), jnp.float32)
            a10 = jnp.zeros((16,), jnp.float32)
            a11 = jnp.zeros((16,), jnp.float32)
            for j in range(16):
                r = ch * 16 + j
                u = [urows_v[r, pl.ds(16 * t, 16)] for t in range(8)]
                v = [irows_v[r, pl.ds(16 * t, 16)] for t in range(8)]
                s00 = jnp.sum(u[0] * v[0] + u[1] * v[1] + u[2] * v[2] + u[3] * v[3])
                s01 = jnp.sum(u[0] * v[4] + u[1] * v[5] + u[2] * v[6] + u[3] * v[7])
                s10 = jnp.sum(u[4] * v[0] + u[5] * v[1] + u[6] * v[2] + u[7] * v[3])
                s11 = jnp.sum(u[4] * v[4] + u[5] * v[5] + u[6] * v[6] + u[7] * v[7])
                sel = lane == j
                a00 = jnp.where(sel, s00, a00)
                a01 = jnp.where(sel, s01, a01)
                a10 = jnp.where(sel, s10, a10)
                a11 = jnp.where(sel, s11, a11)
            chg = h * (BPW // 32) + ch
            c = chg // (ICHUNK // 16)
            k = chg % (ICHUNK // 16)
            s = pl.ds(k * 16, 16)
            pu = jnp.bitwise_and(uorig_v[c, s], 1) == 1
            pi = jnp.bitwise_and(iorig_v[c, s], 1) == 1
            res = jnp.where(pu, jnp.where(pi, a11, a10), jnp.where(pi, a01, a00))
            outb_v[pl.ds(chg * 16, 16)] = res
            return 0

        lax.fori_loop(0, BPW // 32, body, 0)

    pltpu.sync_copy(outb_v, out_hbm.at[wid])


@jax.jit
def _mf(u_id, i_id, user_table, item_table):
    n_users, emb = user_table.shape
    n_items, _ = item_table.shape
    mesh = plsc.VectorSubcoreMesh(core_axis_name="c", subcore_axis_name="s")
    f = functools.partial(
        pl.kernel,
        out_type=jax.ShapeDtypeStruct((NW, BPW), jnp.float32),
        mesh=mesh,
        compiler_params=pltpu.CompilerParams(needs_layout_passes=False),
        scratch_types=[
            pltpu.VMEM((NCHUNK, ICHUNK), jnp.int32),
            pltpu.VMEM((NCHUNK, ICHUNK), jnp.int32),
            pltpu.VMEM((NCHUNK, ICHUNK), jnp.int32),
            pltpu.VMEM((NCHUNK, ICHUNK), jnp.int32),
            pltpu.VMEM((BPW // 2, 2 * EMB), jnp.float32),
            pltpu.VMEM((BPW // 2, 2 * EMB), jnp.float32),
            pltpu.VMEM((BPW,), jnp.float32),
            pltpu.SemaphoreType.DMA,
        ],
    )(_mf_body)
    out = f(u_id.reshape(NW, NCHUNK, ICHUNK).astype(jnp.int32),
            i_id.reshape(NW, NCHUNK, ICHUNK).astype(jnp.int32),
            user_table.reshape(n_users // 2, 2 * emb),
            item_table.reshape(n_items // 2, 2 * emb))
    return out.reshape(BATCH)


def kernel(u_id, i_id, user_table, item_table):
    return _mf(u_id, i_id, user_table, item_table)


# speedup vs baseline: 2.2322x; 2.2322x over previous
import functools

import jax
import jax.numpy as jnp
from jax import lax
from jax.experimental import pallas as pl
from jax.experimental.pallas import tpu as pltpu
from jax.experimental.pallas import tpu_sc as plsc

EMB = 64
BATCH = 16384

NC = 2
NS = 16
NW = NC * NS
BPW = BATCH // NW
NG = BPW // 16
ICHUNK = 128
NCHUNK = BPW // ICHUNK


def _mf_body(su_hbm, si_hbm, perm_hbm, ut_hbm, ip_hbm, out_hbm,
             su_v, sidx_v, poff_v, spidx_v, ipair_v, ut_buf, uT_v,
             res_v, tmp_v, spimg, sem_i, sem_a, sem_b):
    cid = lax.axis_index("c")
    sid = lax.axis_index("s")
    wid = sid * NC + cid

    pltpu.sync_copy(su_hbm.at[wid], su_v)
    pltpu.sync_copy(si_hbm.at[wid], poff_v)
    pltpu.sync_copy(perm_hbm.at[wid], tmp_v)

    for k in range(NCHUNK):
        for j in range(ICHUNK // 16):
            s16 = pl.ds(k * ICHUNK + j * 16, 16)
            si16 = poff_v[s16]
            sidx_v[k, pl.ds(j * 16, 16)] = jnp.right_shift(si16, 1)
            spidx_v[k, pl.ds(j * 16, 16)] = tmp_v[s16]
    for g in range(NG):
        sg = pl.ds(g * 16, 16)
        poff_v[sg] = jnp.bitwise_and(poff_v[sg], 1) * 64

    item_cps = []
    for k in range(NCHUNK):
        item_cps.append(pltpu.async_copy(
            ip_hbm.at[sidx_v.at[k]],
            ipair_v.at[pl.ds(k * ICHUNK, ICHUNK)], sem_i))

    zero = jnp.zeros((16,), jnp.float32)

    def zrow(f, _):
        for g in range(NG):
            uT_v[f, pl.ds(g * 16, 16)] = zero
        return 0

    lax.fori_loop(0, EMB, zrow, 0)

    pltpu.sync_copy(uT_v.at[0], spimg.at[pl.ds(sid * 1024, BPW)])
    pltpu.sync_copy(uT_v.at[1], spimg.at[pl.ds(sid * 1024 + BPW, BPW)])

    b_lo = jnp.min(su_v[pl.ds(0, 16)]) >> 7
    b_hi = jnp.max(su_v[pl.ds(BPW - 16, 16)]) >> 7
    nblk = b_hi - b_lo + 1

    def issue(blk, slot, sem):
        blk = jnp.minimum(blk, b_hi)
        col0 = pl.multiple_of(blk * 128, 128)
        return pltpu.async_copy(ut_hbm.at[:, pl.ds(col0, 128)],
                                ut_buf.at[slot], sem)

    def extract(blk, slot, g0):
        col0 = blk * 128
        cend = col0 + 128

        def first_col(g):
            gc = jnp.minimum(g, NG - 1)
            return jnp.min(su_v[pl.ds(gc * 16, 16)])

        def cond(st):
            return st[1]

        def body(st):
            g, _ = st
            gc = jnp.minimum(g, NG - 1)
            sg = pl.ds(gc * 16, 16)
            su_g = su_v[sg]
            m = (su_g >= col0) & (su_g < cend)
            lcol = jnp.clip(su_g - col0, 0, 127)
            for f in range(EMB):
                uv = plsc.load_gather(
                    ut_buf.at[slot], [jnp.full((16,), f, jnp.int32), lcol])
                uT_v[f, sg] = jnp.where(m, uv, uT_v[f, sg])
            adv = (jnp.max(su_g) < cend).astype(jnp.int32)
            g2 = g + adv
            go = (adv == 1) & (g2 < NG) & (first_col(g2) < cend)
            return g2, go
        g0 = jnp.minimum(g0, NG - 1)
        go0 = first_col(g0) < cend
        gf, _ = lax.while_loop(cond, body, (g0, go0))
        return gf

    cp0 = issue(b_lo, 0, sem_a)
    cp1 = issue(b_lo + 1, 1, sem_b)

    def chunk(tt, g):
        t0 = b_lo + 2 * tt
        pltpu.make_async_copy(ut_hbm.at[:, pl.ds(0, 128)],
                             ut_buf.at[0], sem_a).wait()
        g = extract(t0, 0, g)
        issue(t0 + 2, 0, sem_a)
        pltpu.make_async_copy(ut_hbm.at[:, pl.ds(0, 128)],
                             ut_buf.at[1], sem_b).wait()
        g = extract(jnp.minimum(t0 + 1, b_hi), 1, g)
        issue(t0 + 3, 1, sem_b)
        return g

    nit = (nblk + 1) >> 1
    lax.fori_loop(0, nit, chunk, jnp.int32(0))

    pltpu.make_async_copy(ut_hbm.at[:, pl.ds(0, 128)], ut_buf.at[0], sem_a).wait()
    pltpu.make_async_copy(ut_hbm.at[:, pl.ds(0, 128)], ut_buf.at[1], sem_b).wait()

    for cp in item_cps:
        cp.wait()

    def dot(g, _):
        sg = pl.ds(g * 16, 16)
        rows = g * 16 + lax.iota(jnp.int32, 16)
        poff = poff_v[sg]
        acc = uT_v[0, sg] * plsc.load_gather(ipair_v, [rows, poff])
        for f in range(1, EMB):
            acc += uT_v[f, sg] * plsc.load_gather(ipair_v, [rows, poff + f])
        res_v[sg] = acc
        return 0

    lax.fori_loop(0, NG, dot, 0)

    plsc.subcore_barrier()
    for k in range(NCHUNK):
        pltpu.sync_copy(res_v.at[pl.ds(k * ICHUNK, ICHUNK)],
                        spimg.at[spidx_v.at[k]])
    plsc.subcore_barrier()
    pltpu.sync_copy(spimg.at[pl.ds(sid * (BATCH // NS), BATCH // NS)],
                    out_hbm.at[cid, sid])


@jax.jit
def _mf(u_id, i_id, user_table, item_table):
    n_items, emb = item_table.shape
    u_id = u_id.astype(jnp.int32)
    i_id = i_id.astype(jnp.int32)
    skey = jnp.sort((u_id >> 7) * BATCH + jnp.arange(BATCH, dtype=jnp.int32))
    perm = jnp.bitwise_and(skey, BATCH - 1)
    su = u_id[perm]
    si = i_id[perm]

    mesh = plsc.VectorSubcoreMesh(core_axis_name="c", subcore_axis_name="s")
    f = functools.partial(
        pl.kernel,
        out_type=jax.ShapeDtypeStruct((NC, NS, BATCH // NS), jnp.float32),
        mesh=mesh,
        compiler_params=pltpu.CompilerParams(needs_layout_passes=False),
        scratch_types=[
            pltpu.VMEM((BPW,), jnp.int32),
            pltpu.VMEM((NCHUNK, ICHUNK), jnp.int32),
            pltpu.VMEM((BPW,), jnp.int32),
            pltpu.VMEM((NCHUNK, ICHUNK), jnp.int32),
            pltpu.VMEM((BPW, 2 * EMB), jnp.float32),
            pltpu.VMEM((2, EMB, 128), jnp.float32),
            pltpu.VMEM((EMB, BPW), jnp.float32),
            pltpu.VMEM((BPW,), jnp.float32),
            pltpu.VMEM((BPW,), jnp.int32),
            pltpu.VMEM_SHARED((BATCH,), jnp.float32),
            pltpu.SemaphoreType.DMA,
            pltpu.SemaphoreType.DMA,
            pltpu.SemaphoreType.DMA,
        ],
    )(_mf_body)
    out = f(su.reshape(NW, BPW),
            si.reshape(NW, BPW),
            perm.reshape(NW, BPW),
            jnp.transpose(user_table),
            item_table.reshape(n_items // 2, 2 * emb))
    return (out[0] + out[1]).reshape(BATCH)


def kernel(u_id, i_id, user_table, item_table):
    return _mf(u_id, i_id, user_table, item_table)


# speedup vs baseline: 2.3101x; 1.0349x over previous
import functools

import jax
import jax.numpy as jnp
from jax import lax
from jax.experimental import pallas as pl
from jax.experimental.pallas import tpu as pltpu
from jax.experimental.pallas import tpu_sc as plsc

EMB = 64
BATCH = 16384

NC = 2
NS = 16
NW = NC * NS
BPW = BATCH // NW
NG = BPW // 16
ICHUNK = 128
NCHUNK = BPW // ICHUNK


def _mf_body(uid_hbm, iid_hbm, perm_hbm, ut_hbm, ip_hbm, out_hbm,
             su_v, sidx_v, poff_v, spidx_v, uidf_v, iidf_v, ipair_v, ut_buf,
             uT_v, res_v, tmp_v, spimg, sem_i, sem_a, sem_b):
    cid = lax.axis_index("c")
    sid = lax.axis_index("s")
    wid = cid * NS + sid

    pltpu.sync_copy(uid_hbm, uidf_v)
    pltpu.sync_copy(iid_hbm, iidf_v)
    pltpu.sync_copy(perm_hbm.at[wid], tmp_v)

    for k in range(NCHUNK):
        for j in range(ICHUNK // 16):
            s16 = pl.ds(k * ICHUNK + j * 16, 16)
            pv = tmp_v[s16]
            su16 = plsc.load_gather(uidf_v, [pv])
            si16 = plsc.load_gather(iidf_v, [pv])
            su_v[s16] = su16
            spidx_v[k, pl.ds(j * 16, 16)] = pv
            sidx_v[k, pl.ds(j * 16, 16)] = jnp.right_shift(si16, 1)
            poff_v[s16] = jnp.bitwise_and(si16, 1) * 64

    item_cps = []
    for k in range(NCHUNK // 2):
        item_cps.append(pltpu.async_copy(
            ip_hbm.at[sidx_v.at[k]],
            ipair_v.at[pl.ds(k * ICHUNK, ICHUNK)], sem_i))

    zero = jnp.zeros((16,), jnp.float32)

    def zrow(f, _):
        for g in range(NG):
            uT_v[f, pl.ds(g * 16, 16)] = zero
        return 0

    lax.fori_loop(0, EMB, zrow, 0)

    pltpu.sync_copy(uT_v.at[0], spimg.at[pl.ds(sid * 1024, BPW)])
    pltpu.sync_copy(uT_v.at[1], spimg.at[pl.ds(sid * 1024 + BPW, BPW)])

    b_lo = jnp.min(su_v[pl.ds(0, 16)]) >> 7
    b_hi = jnp.max(su_v[pl.ds(BPW - 16, 16)]) >> 7
    nblk = b_hi - b_lo + 1

    def issue(blk, slot, sem):
        blk = jnp.minimum(blk, b_hi)
        col0 = pl.multiple_of(blk * 128, 128)
        return pltpu.async_copy(ut_hbm.at[:, pl.ds(col0, 128)],
                                ut_buf.at[slot], sem)

    def extract(blk, slot, g0):
        col0 = blk * 128
        cend = col0 + 128

        def first_col(g):
            gc = jnp.minimum(g, NG - 1)
            return jnp.min(su_v[pl.ds(gc * 16, 16)])

        def cond(st):
            return st[1]

        def body(st):
            g, _ = st
            gc = jnp.minimum(g, NG - 1)
            sg = pl.ds(gc * 16, 16)
            su_g = su_v[sg]
            m = (su_g >= col0) & (su_g < cend)
            lcol = jnp.clip(su_g - col0, 0, 127)
            for f in range(EMB):
                uv = plsc.load_gather(
                    ut_buf.at[slot], [jnp.full((16,), f, jnp.int32), lcol])
                uT_v[f, sg] = jnp.where(m, uv, uT_v[f, sg])
            adv = (jnp.max(su_g) < cend).astype(jnp.int32)
            g2 = g + adv
            go = (adv == 1) & (g2 < NG) & (first_col(g2) < cend)
            return g2, go
        g0 = jnp.minimum(g0, NG - 1)
        go0 = first_col(g0) < cend
        gf, _ = lax.while_loop(cond, body, (g0, go0))
        return gf

    cp0 = issue(b_lo, 0, sem_a)
    cp1 = issue(b_lo + 1, 1, sem_b)

    def chunk(tt, g):
        t0 = b_lo + 2 * tt
        pltpu.make_async_copy(ut_hbm.at[:, pl.ds(0, 128)],
                             ut_buf.at[0], sem_a).wait()
        g = extract(t0, 0, g)
        issue(t0 + 2, 0, sem_a)
        pltpu.make_async_copy(ut_hbm.at[:, pl.ds(0, 128)],
                             ut_buf.at[1], sem_b).wait()
        g = extract(jnp.minimum(t0 + 1, b_hi), 1, g)
        issue(t0 + 3, 1, sem_b)
        return g

    nit = (nblk + 1) >> 1
    lax.fori_loop(0, nit, chunk, jnp.int32(0))

    pltpu.make_async_copy(ut_hbm.at[:, pl.ds(0, 128)], ut_buf.at[0], sem_a).wait()
    pltpu.make_async_copy(ut_hbm.at[:, pl.ds(0, 128)], ut_buf.at[1], sem_b).wait()

    def dot(g, _):
        sg = pl.ds(g * 16, 16)
        rows = jnp.bitwise_and(g, (NG // 2) - 1) * 16 + lax.iota(jnp.int32, 16)
        poff = poff_v[sg]
        acc = uT_v[0, sg] * plsc.load_gather(ipair_v, [rows, poff])
        for f in range(1, EMB):
            acc += uT_v[f, sg] * plsc.load_gather(ipair_v, [rows, poff + f])
        res_v[sg] = acc
        return 0

    for cp in item_cps:
        cp.wait()
    lax.fori_loop(0, NG // 2, dot, 0)

    item_cps2 = []
    for k in range(NCHUNK // 2, NCHUNK):
        item_cps2.append(pltpu.async_copy(
            ip_hbm.at[sidx_v.at[k]],
            ipair_v.at[pl.ds((k - NCHUNK // 2) * ICHUNK, ICHUNK)], sem_i))
    for cp in item_cps2:
        cp.wait()
    lax.fori_loop(NG // 2, NG, dot, 0)

    plsc.subcore_barrier()
    for k in range(NCHUNK):
        pltpu.sync_copy(res_v.at[pl.ds(k * ICHUNK, ICHUNK)],
                        spimg.at[spidx_v.at[k]])
    plsc.subcore_barrier()
    pltpu.sync_copy(spimg.at[pl.ds(sid * (BATCH // NS), BATCH // NS)],
                    out_hbm.at[cid, sid])


@jax.jit
def _mf(u_id, i_id, user_table, item_table):
    n_items, emb = item_table.shape
    u_id = u_id.astype(jnp.int32)
    i_id = i_id.astype(jnp.int32)
    skey = jnp.sort((u_id >> 7) * BATCH + jnp.arange(BATCH, dtype=jnp.int32))
    perm = jnp.bitwise_and(skey, BATCH - 1)

    mesh = plsc.VectorSubcoreMesh(core_axis_name="c", subcore_axis_name="s")
    f = functools.partial(
        pl.kernel,
        out_type=jax.ShapeDtypeStruct((NC, NS, BATCH // NS), jnp.float32),
        mesh=mesh,
        compiler_params=pltpu.CompilerParams(needs_layout_passes=False),
        scratch_types=[
            pltpu.VMEM((BPW,), jnp.int32),
            pltpu.VMEM((NCHUNK, ICHUNK), jnp.int32),
            pltpu.VMEM((BPW,), jnp.int32),
            pltpu.VMEM((NCHUNK, ICHUNK), jnp.int32),
            pltpu.VMEM((BATCH,), jnp.int32),
            pltpu.VMEM((BATCH,), jnp.int32),
            pltpu.VMEM((BPW // 2, 2 * EMB), jnp.float32),
            pltpu.VMEM((2, EMB, 128), jnp.float32),
            pltpu.VMEM((EMB, BPW), jnp.float32),
            pltpu.VMEM((BPW,), jnp.float32),
            pltpu.VMEM((BPW,), jnp.int32),
            pltpu.VMEM_SHARED((BATCH,), jnp.float32),
            pltpu.SemaphoreType.DMA,
            pltpu.SemaphoreType.DMA,
            pltpu.SemaphoreType.DMA,
        ],
    )(_mf_body)
    out = f(u_id,
            i_id,
            perm.reshape(NW, BPW),
            jnp.transpose(user_table),
            item_table.reshape(n_items // 2, 2 * emb))
    return (out[0] + out[1]).reshape(BATCH)


def kernel(u_id, i_id, user_table, item_table):
    return _mf(u_id, i_id, user_table, item_table)


# speedup vs baseline: 2.9804x; 1.2901x over previous
import functools

import jax
import jax.numpy as jnp
from jax import lax
from jax.experimental import pallas as pl
from jax.experimental.pallas import tpu as pltpu
from jax.experimental.pallas import tpu_sc as plsc

EMB = 64
BATCH = 16384

NC = 2
NS = 16
NW = NC * NS
BPW = BATCH // NW
NG = BPW // 16
ICHUNK = 128
NCHUNK = BPW // ICHUNK


def _mf_body(uid_hbm, iid_hbm, perm_hbm, ut_hbm, ip_hbm, out_hbm,
             su_v, sidx_v, poff_v, spidx_v, uidf_v, iidf_v, ipair_v, ut_buf,
             uT_v, res_v, tmp_v, spimg, sem_i, sem_a, sem_b, sem_c, sem_d):
    cid = lax.axis_index("c")
    sid = lax.axis_index("s")
    wid = cid * NS + sid

    pltpu.sync_copy(uid_hbm, uidf_v)
    pltpu.sync_copy(iid_hbm, iidf_v)
    pltpu.sync_copy(perm_hbm.at[wid], tmp_v)

    for k in range(NCHUNK):
        for j in range(ICHUNK // 16):
            s16 = pl.ds(k * ICHUNK + j * 16, 16)
            pv = tmp_v[s16]
            su16 = plsc.load_gather(uidf_v, [pv])
            si16 = plsc.load_gather(iidf_v, [pv])
            su_v[s16] = su16
            spidx_v[k, pl.ds(j * 16, 16)] = pv
            sidx_v[k, pl.ds(j * 16, 16)] = jnp.right_shift(si16, 1)
            poff_v[s16] = jnp.bitwise_and(si16, 1) * 64

    cp_item = pltpu.async_copy(ip_hbm.at[sidx_v.at[0]], ipair_v, sem_i)

    zero = jnp.zeros((16,), jnp.float32)

    def zrow(f, _):
        for g in range(NG):
            uT_v[f, pl.ds(g * 16, 16)] = zero
        return 0

    lax.fori_loop(0, EMB, zrow, 0)

    pltpu.sync_copy(uT_v.at[0], spimg.at[pl.ds(sid * 1024, BPW)])
    pltpu.sync_copy(uT_v.at[1], spimg.at[pl.ds(sid * 1024 + BPW, BPW)])

    b_lo = jnp.min(su_v[pl.ds(0, 16)]) >> 7
    b_hi = jnp.max(su_v[pl.ds(BPW - 16, 16)]) >> 7
    nblk = b_hi - b_lo + 1

    def issue(blk, slot, sem):
        blk = jnp.minimum(blk, b_hi)
        col0 = pl.multiple_of(blk * 128, 128)
        return pltpu.async_copy(ut_hbm.at[:, pl.ds(col0, 128)],
                                ut_buf.at[slot], sem)

    def extract(blk, slot, g0):
        col0 = blk * 128
        cend = col0 + 128

        def first_col(g):
            gc = jnp.minimum(g, NG - 1)
            return jnp.min(su_v[pl.ds(gc * 16, 16)])

        def cond(st):
            return st[1]

        def body(st):
            g, _ = st
            gc = jnp.minimum(g, NG - 1)
            sg = pl.ds(gc * 16, 16)
            su_g = su_v[sg]
            m = (su_g >= col0) & (su_g < cend)
            lcol = jnp.clip(su_g - col0, 0, 127)
            for f in range(EMB):
                uv = plsc.load_gather(
                    ut_buf.at[slot], [jnp.full((16,), f, jnp.int32), lcol])
                uT_v[f, sg] = jnp.where(m, uv, uT_v[f, sg])
            adv = (jnp.max(su_g) < cend).astype(jnp.int32)
            g2 = g + adv
            go = (adv == 1) & (g2 < NG) & (first_col(g2) < cend)
            return g2, go
        g0 = jnp.minimum(g0, NG - 1)
        go0 = first_col(g0) < cend
        gf, _ = lax.while_loop(cond, body, (g0, go0))
        return gf

    sems = [sem_a, sem_b, sem_c, sem_d]
    for j in range(4):
        issue(b_lo + j, j, sems[j])

    def chunk(tt, g):
        t0 = b_lo + 4 * tt
        for j in range(4):
            pltpu.make_async_copy(ut_hbm.at[:, pl.ds(0, 128)],
                                 ut_buf.at[j], sems[j]).wait()
            g = extract(jnp.minimum(t0 + j, b_hi), j, g)
            issue(t0 + j + 4, j, sems[j])
        return g

    nit = (nblk + 3) >> 2
    lax.fori_loop(0, nit, chunk, jnp.int32(0))

    for j in range(4):
        pltpu.make_async_copy(ut_hbm.at[:, pl.ds(0, 128)],
                             ut_buf.at[j], sems[j]).wait()

    def dot(g, _):
        sg = pl.ds(g * 16, 16)
        rows = jnp.bitwise_and(g, (ICHUNK // 16) - 1) * 16 + lax.iota(jnp.int32, 16)
        poff = poff_v[sg]
        acc = uT_v[0, sg] * plsc.load_gather(ipair_v, [rows, poff])
        for f in range(1, EMB):
            acc += uT_v[f, sg] * plsc.load_gather(ipair_v, [rows, poff + f])
        res_v[sg] = acc
        return 0

    for k in range(NCHUNK):
        cp_item.wait()
        lax.fori_loop(k * (ICHUNK // 16), (k + 1) * (ICHUNK // 16), dot, 0)
        if k + 1 < NCHUNK:
            cp_item = pltpu.async_copy(
                ip_hbm.at[sidx_v.at[k + 1]], ipair_v, sem_i)

    plsc.subcore_barrier()
    for k in range(NCHUNK):
        pltpu.sync_copy(res_v.at[pl.ds(k * ICHUNK, ICHUNK)],
                        spimg.at[spidx_v.at[k]])
    plsc.subcore_barrier()
    pltpu.sync_copy(spimg.at[pl.ds(sid * (BATCH // NS), BATCH // NS)],
                    out_hbm.at[cid, sid])


@jax.jit
def _mf(u_id, i_id, user_table, item_table):
    n_items, emb = item_table.shape
    u_id = u_id.astype(jnp.int32)
    i_id = i_id.astype(jnp.int32)
    skey = jnp.sort((u_id >> 7) * BATCH + jnp.arange(BATCH, dtype=jnp.int32))
    perm = jnp.bitwise_and(skey, BATCH - 1)

    mesh = plsc.VectorSubcoreMesh(core_axis_name="c", subcore_axis_name="s")
    f = functools.partial(
        pl.kernel,
        out_type=jax.ShapeDtypeStruct((NC, NS, BATCH // NS), jnp.float32),
        mesh=mesh,
        compiler_params=pltpu.CompilerParams(needs_layout_passes=False),
        scratch_types=[
            pltpu.VMEM((BPW,), jnp.int32),
            pltpu.VMEM((NCHUNK, ICHUNK), jnp.int32),
            pltpu.VMEM((BPW,), jnp.int32),
            pltpu.VMEM((NCHUNK, ICHUNK), jnp.int32),
            pltpu.VMEM((BATCH,), jnp.int32),
            pltpu.VMEM((BATCH,), jnp.int32),
            pltpu.VMEM((ICHUNK, 2 * EMB), jnp.float32),
            pltpu.VMEM((4, EMB, 128), jnp.float32),
            pltpu.VMEM((EMB, BPW), jnp.float32),
            pltpu.VMEM((BPW,), jnp.float32),
            pltpu.VMEM((BPW,), jnp.int32),
            pltpu.VMEM_SHARED((BATCH,), jnp.float32),
            pltpu.SemaphoreType.DMA,
            pltpu.SemaphoreType.DMA,
            pltpu.SemaphoreType.DMA,
            pltpu.SemaphoreType.DMA,
            pltpu.SemaphoreType.DMA,
        ],
    )(_mf_body)
    out = f(u_id,
            i_id,
            perm.reshape(NW, BPW),
            jnp.transpose(user_table),
            item_table.reshape(n_items // 2, 2 * emb))
    return (out[0] + out[1]).reshape(BATCH)


def kernel(u_id, i_id, user_table, item_table):
    return _mf(u_id, i_id, user_table, item_table)


# speedup vs baseline: 2.9988x; 1.0062x over previous
import functools

import jax
import jax.numpy as jnp
from jax import lax
from jax.experimental import pallas as pl
from jax.experimental.pallas import tpu as pltpu
from jax.experimental.pallas import tpu_sc as plsc

EMB = 64
BATCH = 16384
N_USERS = 1000000
WINW = 256

NC = 2
NS = 16
NW = NC * NS
BPW = BATCH // NW
NG = BPW // 16
ICHUNK = 128
NCHUNK = BPW // ICHUNK


def _mf_body(uid_hbm, iid_hbm, perm_hbm, ut_hbm, ip_hbm, out_hbm,
             su_v, sidx_v, poff_v, spidx_v, idf_v, ipair_v, ut_buf,
             uT_v, res_v, tmp_v, spimg, sem_i, sem_a, sem_b, sem_c):
    cid = lax.axis_index("c")
    sid = lax.axis_index("s")
    wid = cid * NS + sid

    pltpu.sync_copy(uid_hbm, idf_v)
    pltpu.sync_copy(perm_hbm.at[wid], tmp_v)

    for k in range(NCHUNK):
        for j in range(ICHUNK // 16):
            s16 = pl.ds(k * ICHUNK + j * 16, 16)
            pv = tmp_v[s16]
            su_v[s16] = plsc.load_gather(idf_v, [pv])
            spidx_v[k, pl.ds(j * 16, 16)] = pv

    pltpu.sync_copy(iid_hbm, idf_v)
    for k in range(NCHUNK):
        for j in range(ICHUNK // 16):
            s16 = pl.ds(k * ICHUNK + j * 16, 16)
            si16 = plsc.load_gather(idf_v, [tmp_v[s16]])
            sidx_v[k, pl.ds(j * 16, 16)] = jnp.right_shift(si16, 1)
            poff_v[s16] = jnp.bitwise_and(si16, 1) * 64

    cp_item = pltpu.async_copy(ip_hbm.at[sidx_v.at[0]], ipair_v, sem_i)

    zero = jnp.zeros((16,), jnp.float32)

    def zrow(f, _):
        for g in range(NG):
            uT_v[f, pl.ds(g * 16, 16)] = zero
        return 0

    lax.fori_loop(0, EMB, zrow, 0)

    pltpu.sync_copy(uT_v.at[0], spimg.at[pl.ds(sid * 1024, BPW)])
    pltpu.sync_copy(uT_v.at[1], spimg.at[pl.ds(sid * 1024 + BPW, BPW)])

    b_lo = jnp.min(su_v[pl.ds(0, 16)]) >> 7
    b_hi = jnp.max(su_v[pl.ds(BPW - 16, 16)]) >> 7
    nblk = b_hi - b_lo + 1
    nwin = (nblk + 1) >> 1
    maxcol0 = (((N_USERS + 127) // 128) - 2) * 128

    def wcol0(w):
        w = jnp.minimum(w, nwin - 1)
        return jnp.minimum((b_lo + 2 * w) * 128, maxcol0)

    def issue(w, slot, sem):
        col0 = pl.multiple_of(wcol0(w), 128)
        return pltpu.async_copy(ut_hbm.at[:, pl.ds(col0, WINW)],
                                ut_buf.at[slot], sem)

    def extract(w, slot, g0):
        col0 = wcol0(w)
        cend = col0 + WINW

        def first_col(g):
            gc = jnp.minimum(g, NG - 1)
            return jnp.min(su_v[pl.ds(gc * 16, 16)])

        def cond(st):
            return st[1]

        def body(st):
            g, _ = st
            gc = jnp.minimum(g, NG - 1)
            sg = pl.ds(gc * 16, 16)
            su_g = su_v[sg]
            m = (su_g >= col0) & (su_g < cend)
            lcol = jnp.clip(su_g - col0, 0, WINW - 1)
            for f in range(EMB):
                uv = plsc.load_gather(
                    ut_buf.at[slot], [jnp.full((16,), f, jnp.int32), lcol])
                uT_v[f, sg] = jnp.where(m, uv, uT_v[f, sg])
            adv = (jnp.max(su_g) < cend).astype(jnp.int32)
            g2 = g + adv
            go = (adv == 1) & (g2 < NG) & (first_col(g2) < cend)
            return g2, go
        g0 = jnp.minimum(g0, NG - 1)
        go0 = first_col(g0) < cend
        gf, _ = lax.while_loop(cond, body, (g0, go0))
        return gf

    sems = [sem_a, sem_b, sem_c]
    for j in range(3):
        issue(jnp.int32(j), j, sems[j])

    def chunk(tt, g):
        w0 = 3 * tt
        for j in range(3):
            pltpu.make_async_copy(ut_hbm.at[:, pl.ds(0, WINW)],
                                 ut_buf.at[j], sems[j]).wait()
            g = extract(w0 + j, j, g)
            issue(w0 + j + 3, j, sems[j])
        return g

    nit = (nwin + 2) // 3
    lax.fori_loop(0, nit, chunk, jnp.int32(0))

    for j in range(3):
        pltpu.make_async_copy(ut_hbm.at[:, pl.ds(0, WINW)],
                             ut_buf.at[j], sems[j]).wait()

    def dot(g, _):
        sg = pl.ds(g * 16, 16)
        rows = jnp.bitwise_and(g, (ICHUNK // 16) - 1) * 16 + lax.iota(jnp.int32, 16)
        poff = poff_v[sg]
        acc = uT_v[0, sg] * plsc.load_gather(ipair_v, [rows, poff])
        for f in range(1, EMB):
            acc += uT_v[f, sg] * plsc.load_gather(ipair_v, [rows, poff + f])
        res_v[sg] = acc
        return 0

    for k in range(NCHUNK):
        cp_item.wait()
        lax.fori_loop(k * (ICHUNK // 16), (k + 1) * (ICHUNK // 16), dot, 0)
        if k + 1 < NCHUNK:
            cp_item = pltpu.async_copy(
                ip_hbm.at[sidx_v.at[k + 1]], ipair_v, sem_i)

    plsc.subcore_barrier()
    for k in range(NCHUNK):
        pltpu.sync_copy(res_v.at[pl.ds(k * ICHUNK, ICHUNK)],
                        spimg.at[spidx_v.at[k]])
    plsc.subcore_barrier()
    pltpu.sync_copy(spimg.at[pl.ds(sid * (BATCH // NS), BATCH // NS)],
                    out_hbm.at[cid, sid])


@jax.jit
def _mf(u_id, i_id, user_table, item_table):
    n_items, emb = item_table.shape
    u_id = u_id.astype(jnp.int32)
    i_id = i_id.astype(jnp.int32)
    skey = jnp.sort((u_id >> 7) * BATCH + jnp.arange(BATCH, dtype=jnp.int32))
    perm = jnp.bitwise_and(skey, BATCH - 1)

    mesh = plsc.VectorSubcoreMesh(core_axis_name="c", subcore_axis_name="s")
    f = functools.partial(
        pl.kernel,
        out_type=jax.ShapeDtypeStruct((NC, NS, BATCH // NS), jnp.float32),
        mesh=mesh,
        compiler_params=pltpu.CompilerParams(needs_layout_passes=False),
        scratch_types=[
            pltpu.VMEM((BPW,), jnp.int32),
            pltpu.VMEM((NCHUNK, ICHUNK), jnp.int32),
            pltpu.VMEM((BPW,), jnp.int32),
            pltpu.VMEM((NCHUNK, ICHUNK), jnp.int32),
            pltpu.VMEM((BATCH,), jnp.int32),
            pltpu.VMEM((ICHUNK, 2 * EMB), jnp.float32),
            pltpu.VMEM((3, EMB, WINW), jnp.float32),
            pltpu.VMEM((EMB, BPW), jnp.float32),
            pltpu.VMEM((BPW,), jnp.float32),
            pltpu.VMEM((BPW,), jnp.int32),
            pltpu.VMEM_SHARED((BATCH,), jnp.float32),
            pltpu.SemaphoreType.DMA,
            pltpu.SemaphoreType.DMA,
            pltpu.SemaphoreType.DMA,
            pltpu.SemaphoreType.DMA,
        ],
    )(_mf_body)
    out = f(u_id,
            i_id,
            perm.reshape(NW, BPW),
            jnp.transpose(user_table),
            item_table.reshape(n_items // 2, 2 * emb))
    return (out[0] + out[1]).reshape(BATCH)


def kernel(u_id, i_id, user_table, item_table):
    return _mf(u_id, i_id, user_table, item_table)
